# baseline (speedup 1.0000x reference)
import functools

import jax
import jax.numpy as jnp
from jax import lax
from jax.experimental import pallas as pl
from jax.experimental.pallas import tpu as pltpu

_ALPHA = 0.2


def _prep_kernel(x_ref, wpre_ref, watt_ref,
                 h_ref, Esb_ref, Es2b_ref, rvec_ref):
    h_ref[...] = jnp.dot(x_ref[...], wpre_ref[...],
                         preferred_element_type=jnp.float32)
    hv = h_ref[...]
    n, u = hv.shape
    a = watt_ref[...]
    e_s = lax.dot_general(a[:u, :], hv, (((0,), (1,)), ((), ())),
                          preferred_element_type=jnp.float32)
    e_r = lax.dot_general(a[u:, :], hv, (((0,), (1,)), ((), ())),
                          preferred_element_type=jnp.float32)
    m = jnp.max(e_s)
    ones_row = jnp.ones((1, u), jnp.float32)
    Esb_ref[...] = jnp.exp(e_s - m).reshape(n, 1) * ones_row
    Es2b_ref[...] = jnp.exp(_ALPHA * (e_s - m)).reshape(n, 1) * ones_row
    t = m + e_r
    c = jnp.where(t > 0.0, t, _ALPHA * t)
    rvec_ref[0:1, :] = jnp.exp(e_r + m - c)
    rvec_ref[1:2, :] = jnp.exp(_ALPHA * (e_r + m) - c)
    rvec_ref[2:8, :] = jnp.zeros((6, n), jnp.float32)


def _gat_kernel(nblk, adj_ref, h_ref, Esb_ref, Es2b_ref, rvec_ref, out_ref,
                p_buf):
    j = pl.program_id(0)
    n, rb = adj_ref.shape
    Esb = Esb_ref[...]
    Es2b = Es2b_ref[...]
    eye = (lax.broadcasted_iota(jnp.int32, (128, 128), 0)
           == lax.broadcasted_iota(jnp.int32, (128, 128), 1)
           ).astype(jnp.float32)
    contrib = None
    deltas = []
    for g in range(rb // 128):
        lo = g * 128
        a_g = adj_ref[:, lo:lo + 128]
        Er_g = rvec_ref[0:1, lo:lo + 128]
        Er2_g = rvec_ref[1:2, lo:lo + 128]
        p_g = a_g * jnp.maximum(Esb * Er_g, Es2b * Er2_g)
        p_buf[:, lo:lo + 128] = p_g.astype(jnp.bfloat16)
        cs_g = jnp.sum(p_g, axis=0, keepdims=True)
        row0 = j * rb + lo
        sub = adj_ref[pl.ds(row0, 128), lo:lo + 128]
        msel = jnp.maximum(Esb_ref[pl.ds(row0, 128), :] * Er_g,
                           Es2b_ref[pl.ds(row0, 128), :] * Er2_g)
        dmat = eye * (1.0 - sub) * msel
        cs_g = cs_g + jnp.sum(dmat, axis=0, keepdims=True)
        inv_g = (1.0 / cs_g).reshape(128, 1)
        hs_g = h_ref[pl.ds(row0, 128), :] * inv_g
        c_g = jnp.dot(p_buf[:, lo:lo + 128], hs_g.astype(jnp.bfloat16),
                      preferred_element_type=jnp.float32)
        contrib = c_g if contrib is None else contrib + c_g
        dv_g = jnp.sum(dmat, axis=1, keepdims=True)
        deltas.append((row0, dv_g * hs_g))

    @pl.when(j == 0)
    def _():
        out_ref[...] = contrib

    @pl.when(j != 0)
    def _():
        out_ref[...] = out_ref[...] + contrib

    for row0, dlt in deltas:
        out_ref[pl.ds(row0, 128), :] = out_ref[pl.ds(row0, 128), :] + dlt


def _elu_kernel(acc_ref, out_ref):
    o = acc_ref[...]
    out_ref[...] = jnp.where(o > 0.0, o, jnp.exp(o) - 1.0)


def _build_calls(n, d, units, interpret=False):
    prep = pl.pallas_call(
        _prep_kernel,
        out_shape=[
            jax.ShapeDtypeStruct((n, units), jnp.float32),
            jax.ShapeDtypeStruct((n, units), jnp.float32),
            jax.ShapeDtypeStruct((n, units), jnp.float32),
            jax.ShapeDtypeStruct((8, n), jnp.float32),
        ],
        interpret=interpret,
    )
    rb = 512 if n % 512 == 0 else n
    nblk = n // rb
    main = pl.pallas_call(
        functools.partial(_gat_kernel, nblk),
        grid=(nblk,),
        in_specs=[
            pl.BlockSpec((n, rb), lambda j: (0, j)),
            pl.BlockSpec((n, units), lambda j: (0, 0)),
            pl.BlockSpec((n, units), lambda j: (0, 0)),
            pl.BlockSpec((n, units), lambda j: (0, 0)),
            pl.BlockSpec((8, rb), lambda j: (0, j)),
        ],
        out_specs=pl.BlockSpec((n, units), lambda j: (0, 0)),
        out_shape=jax.ShapeDtypeStruct((n, units), jnp.float32),
        scratch_shapes=[pltpu.VMEM((n, rb), jnp.bfloat16)],
        compiler_params=pltpu.CompilerParams(
            dimension_semantics=("arbitrary",)),
        interpret=interpret,
    )
    elu = pl.pallas_call(
        _elu_kernel,
        out_shape=jax.ShapeDtypeStruct((n, units), jnp.float32),
        interpret=interpret,
    )
    return prep, main, elu


def kernel(x, adj, W_pre, W_att):
    b, n, d = x.shape
    units = W_pre.shape[1]
    prep, main, elu = _build_calls(n, d, units)
    h, Esb, Es2b, rvec = prep(x[0], W_pre, W_att)
    out = elu(main(adj[0], h, Esb, Es2b, rvec))
    return out[None]


# speedup vs baseline: 1.0638x; 1.0638x over previous
import functools

import jax
import jax.numpy as jnp
from jax import lax
from jax.experimental import pallas as pl
from jax.experimental.pallas import tpu as pltpu

_ALPHA = 0.2


def _prep_kernel(x_ref, wpre_ref, watt_ref,
                 h_ref, Esb_ref, Es2b_ref, rvec_ref):
    h_ref[...] = jnp.dot(x_ref[...], wpre_ref[...],
                         preferred_element_type=jnp.float32)
    hv = h_ref[...]
    n, u = hv.shape
    a = watt_ref[...]
    e_s = lax.dot_general(a[:u, :], hv, (((0,), (1,)), ((), ())),
                          preferred_element_type=jnp.float32)
    e_r = lax.dot_general(a[u:, :], hv, (((0,), (1,)), ((), ())),
                          preferred_element_type=jnp.float32)
    m = jnp.max(e_s)
    ones_row = jnp.ones((1, u), jnp.float32)
    Esb_ref[...] = jnp.exp(e_s - m).reshape(n, 1) * ones_row
    Es2b_ref[...] = jnp.exp(_ALPHA * (e_s - m)).reshape(n, 1) * ones_row
    t = m + e_r
    c = jnp.where(t > 0.0, t, _ALPHA * t)
    rvec_ref[0:1, :] = jnp.exp(e_r + m - c)
    rvec_ref[1:2, :] = jnp.exp(_ALPHA * (e_r + m) - c)
    rvec_ref[2:8, :] = jnp.zeros((6, n), jnp.float32)


def _gat_kernel(nblk, adj_ref, h_ref, Esb_ref, Es2b_ref, rvec_ref, out_ref,
                p_buf):
    j = pl.program_id(0)
    n, rb = adj_ref.shape
    Esb = Esb_ref[...]
    Es2b = Es2b_ref[...]
    eye = (lax.broadcasted_iota(jnp.int32, (128, 128), 0)
           == lax.broadcasted_iota(jnp.int32, (128, 128), 1)
           ).astype(jnp.float32)
    contrib = None
    deltas = []
    for g in range(rb // 128):
        lo = g * 128
        a_g = adj_ref[:, lo:lo + 128]
        Er_g = rvec_ref[0:1, lo:lo + 128]
        Er2_g = rvec_ref[1:2, lo:lo + 128]
        p_g = a_g * jnp.maximum(Esb * Er_g, Es2b * Er2_g)
        p_buf[:, lo:lo + 128] = p_g.astype(jnp.bfloat16)
        cs_g = jnp.sum(p_g, axis=0, keepdims=True)
        row0 = j * rb + lo
        sub = adj_ref[pl.ds(row0, 128), lo:lo + 128]
        msel = jnp.maximum(Esb_ref[pl.ds(row0, 128), :] * Er_g,
                           Es2b_ref[pl.ds(row0, 128), :] * Er2_g)
        dmat = eye * (1.0 - sub) * msel
        cs_g = cs_g + jnp.sum(dmat, axis=0, keepdims=True)
        inv_g = (1.0 / cs_g).reshape(128, 1)
        hs_g = h_ref[pl.ds(row0, 128), :] * inv_g
        c_g = jnp.dot(p_buf[:, lo:lo + 128], hs_g.astype(jnp.bfloat16),
                      preferred_element_type=jnp.float32)
        contrib = c_g if contrib is None else contrib + c_g
        dv_g = jnp.sum(dmat, axis=1, keepdims=True)
        deltas.append((row0, dv_g * hs_g))

    @pl.when(j == 0)
    def _():
        out_ref[...] = contrib

    @pl.when(j != 0)
    def _():
        out_ref[...] = out_ref[...] + contrib

    for row0, dlt in deltas:
        out_ref[pl.ds(row0, 128), :] = out_ref[pl.ds(row0, 128), :] + dlt

    @pl.when(j == nblk - 1)
    def _():
        o = out_ref[...]
        out_ref[...] = jnp.where(o > 0.0, o, jnp.exp(o) - 1.0)


def _build_calls(n, d, units):
    prep = pl.pallas_call(
        _prep_kernel,
        out_shape=[
            jax.ShapeDtypeStruct((n, units), jnp.float32),
            jax.ShapeDtypeStruct((n, units), jnp.float32),
            jax.ShapeDtypeStruct((n, units), jnp.float32),
            jax.ShapeDtypeStruct((8, n), jnp.float32),
        ],
    )
    rb = 512 if n % 512 == 0 else n
    nblk = n // rb
    main = pl.pallas_call(
        functools.partial(_gat_kernel, nblk),
        grid=(nblk,),
        in_specs=[
            pl.BlockSpec((n, rb), lambda j: (0, j)),
            pl.BlockSpec((n, units), lambda j: (0, 0)),
            pl.BlockSpec((n, units), lambda j: (0, 0)),
            pl.BlockSpec((n, units), lambda j: (0, 0)),
            pl.BlockSpec((8, rb), lambda j: (0, j)),
        ],
        out_specs=pl.BlockSpec((n, units), lambda j: (0, 0)),
        out_shape=jax.ShapeDtypeStruct((n, units), jnp.float32),
        scratch_shapes=[pltpu.VMEM((n, rb), jnp.bfloat16)],
        compiler_params=pltpu.CompilerParams(
            dimension_semantics=("arbitrary",)),
    )
    return prep, main


def kernel(x, adj, W_pre, W_att):
    b, n, d = x.shape
    units = W_pre.shape[1]
    prep, main = _build_calls(n, d, units)
    h, Esb, Es2b, rvec = prep(x[0], W_pre, W_att)
    out = main(adj[0], h, Esb, Es2b, rvec)
    return out[None]
